# baseline (speedup 1.0000x reference)
import functools

import jax
import jax.numpy as jnp
from jax import lax
from jax.experimental import pallas as pl
from jax.experimental.pallas import tpu as pltpu
from jax.experimental.pallas import tpu_sc as plsc

_NC = 2
_NS = 16
_NW = _NC * _NS
_L = 16
_CH = 16
_CHR = 8


@functools.lru_cache(maxsize=None)
def _build_sc_call(B, N):
    NCH = (B // _NW) // _CH
    NCHK = (N + _CHR - 1) // _CHR
    K = (NCHK + _NS - 1) // _NS
    FS = N // _L
    REM = N % _L
    TOFF = N - _L
    ZC = -(-N // (_NS * _CH))
    ZR = ZC * _CH

    mesh = plsc.VectorSubcoreMesh(core_axis_name="c", subcore_axis_name="s")

    def body(p_hbm, t_hbm, c_hbm, out_hbm,
             m_sh, idx_v, pbuf0, pbuf1, mbuf, cbuf, obuf,
             lsem0, lsem1, ssem0, ssem1, msem, csem):
        cid = lax.axis_index("c")
        sid = lax.axis_index("s")
        wid = cid * _NS + sid
        zv = jnp.zeros((_L,), jnp.float32)

        def zfill(j, carry):
            off = j * _L
            for r in range(_CH):
                pbuf0[r, pl.ds(off, _L)] = zv
            return carry
        lax.fori_loop(0, FS, zfill, 0)
        if REM:
            for r in range(_CH):
                pbuf0[r, pl.ds(TOFF, _L)] = zv
        r0 = jnp.minimum(sid * ZR, N - ZR)
        for i in range(ZC):
            pltpu.sync_copy(pbuf0, m_sh.at[pl.ds(r0 + i * _CH, _CH)])
        plsc.subcore_barrier()

        pltpu.sync_copy(t_hbm.at[wid], idx_v)

        rowbase = wid * (NCH * _CH)
        bufs = (pbuf0, pbuf1)
        lsems = (lsem0, lsem1)
        ssems = (ssem0, ssem1)
        ld = [None, None]
        sc = [None, None]
        ld[0] = pltpu.async_copy(p_hbm.at[pl.ds(rowbase, _CH)], pbuf0, lsem0)
        for j in range(NCH):
            b = j % 2
            ld[b].wait()
            if j >= 1:
                sc[1 - b].wait()
            if j + 1 < NCH:
                ld[1 - b] = pltpu.async_copy(
                    p_hbm.at[pl.ds(rowbase + (j + 1) * _CH, _CH)],
                    bufs[1 - b], lsems[1 - b])
            sc[b] = pltpu.async_copy(bufs[b], m_sh.at[idx_v.at[j]],
                                     ssems[b], add=True)
        sc[(NCH - 1) % 2].wait()
        plsc.subcore_barrier()

        lanes = lax.iota(jnp.int32, _L)
        tmask = lanes >= (_L - REM)
        acc = zv
        for k in range(K):
            ckid = sid + _NS * k
            ck = jnp.minimum(ckid, NCHK - 1)
            rr = ck * _CHR
            cm = pltpu.async_copy(m_sh.at[pl.ds(rr, _CHR)], mbuf, msem)
            cc = pltpu.async_copy(c_hbm.at[pl.ds(rr, _CHR)], cbuf, csem)
            cm.wait()
            cc.wait()

            def slice_body(j, a):
                off = j * _L
                for r in range(_CHR):
                    a = a + mbuf[r, pl.ds(off, _L)] * cbuf[r, pl.ds(off, _L)]
                return a
            contrib = lax.fori_loop(0, FS, slice_body, zv)
            if REM:
                t = zv
                for r in range(_CHR):
                    t = t + mbuf[r, pl.ds(TOFF, _L)] * cbuf[r, pl.ds(TOFF, _L)]
                contrib = contrib + jnp.where(tmask, t, zv)
            acc = acc + jnp.where(ckid < NCHK, contrib, zv)

        obuf[...] = acc
        pltpu.sync_copy(obuf, out_hbm.at[wid])

    return pl.kernel(
        body,
        out_type=jax.ShapeDtypeStruct((_NW, _L), jnp.float32),
        mesh=mesh,
        scratch_types=[
            pltpu.VMEM_SHARED((N, N), jnp.float32),
            pltpu.VMEM((NCH, _CH), jnp.int32),
            pltpu.VMEM((_CH, N), jnp.float32),
            pltpu.VMEM((_CH, N), jnp.float32),
            pltpu.VMEM((_CHR, N), jnp.float32),
            pltpu.VMEM((_CHR, N), jnp.float32),
            pltpu.VMEM((_L,), jnp.float32),
            pltpu.SemaphoreType.DMA,
            pltpu.SemaphoreType.DMA,
            pltpu.SemaphoreType.DMA,
            pltpu.SemaphoreType.DMA,
            pltpu.SemaphoreType.DMA,
            pltpu.SemaphoreType.DMA,
        ],
        compiler_params=pltpu.CompilerParams(use_tc_tiling_on_sc=False),
        name="otloss_sc",
    )


def kernel(output_probs, target_class, C):
    B, N = output_probs.shape
    idx3 = target_class.astype(jnp.int32).reshape(_NW, (B // _NW) // _CH, _CH)
    partials = _build_sc_call(B, N)(output_probs, idx3, C)
    return jnp.sum(partials) / B


# speedup vs baseline: 1.3691x; 1.3691x over previous
import functools

import jax
import jax.numpy as jnp
from jax import lax
from jax.experimental import pallas as pl
from jax.experimental.pallas import tpu as pltpu
from jax.experimental.pallas import tpu_sc as plsc

_NC = 2
_NS = 16
_NW = _NC * _NS
_L = 16
_CH = 16


@functools.lru_cache(maxsize=None)
def _build_sc_call(B, N):
    NP = -(-N // 128) * 128
    NCH = (B // _NW) // _CH
    FS = N // _L
    REM = N % _L
    TOFF = N - _L

    mesh = plsc.VectorSubcoreMesh(core_axis_name="c", subcore_axis_name="s")

    def body(p_hbm, t_hbm, c_hbm, out_hbm,
             idx_v, pbuf0, pbuf1, gbuf0, gbuf1, obuf,
             lsem0, lsem1, gsem0, gsem1):
        cid = lax.axis_index("c")
        sid = lax.axis_index("s")
        wid = cid * _NS + sid
        zv = jnp.zeros((_L,), jnp.float32)
        lanes = lax.iota(jnp.int32, _L)
        tmask = lanes >= (_L - REM)

        pltpu.sync_copy(t_hbm.at[wid], idx_v)

        rowbase = wid * (NCH * _CH)
        pbufs = (pbuf0, pbuf1)
        gbufs = (gbuf0, gbuf1)
        lsems = (lsem0, lsem1)
        gsems = (gsem0, gsem1)

        def start(j):
            b = j % 2
            ldj = pltpu.async_copy(
                p_hbm.at[pl.ds(rowbase + j * _CH, _CH)], pbufs[b], lsems[b])
            gdj = pltpu.async_copy(
                c_hbm.at[idx_v.at[j]], gbufs[b], gsems[b])
            return ldj, gdj

        def chunk_dot(pb, gb, acc):
            def fbody(jj, a):
                off = jj * _L
                for r in range(_CH):
                    a = a + pb[r, pl.ds(off, _L)] * gb[r, pl.ds(off, _L)]
                return a
            acc = lax.fori_loop(0, FS, fbody, acc)
            if REM:
                t = zv
                for r in range(_CH):
                    t = t + pb[r, pl.ds(TOFF, _L)] * gb[r, pl.ds(TOFF, _L)]
                acc = acc + jnp.where(tmask, t, zv)
            return acc

        acc = zv
        pend = [None, None]
        pend[0] = start(0)
        for j in range(NCH):
            b = j % 2
            ldj, gdj = pend[b]
            ldj.wait()
            gdj.wait()
            if j + 1 < NCH:
                pend[1 - b] = start(j + 1)
            acc = chunk_dot(pbufs[b], gbufs[b], acc)

        obuf[...] = acc
        pltpu.sync_copy(obuf, out_hbm.at[wid])

    return pl.kernel(
        body,
        out_type=jax.ShapeDtypeStruct((_NW, _L), jnp.float32),
        mesh=mesh,
        scratch_types=[
            pltpu.VMEM((NCH, _CH), jnp.int32),
            pltpu.VMEM((_CH, N), jnp.float32),
            pltpu.VMEM((_CH, N), jnp.float32),
            pltpu.VMEM((_CH, NP), jnp.float32),
            pltpu.VMEM((_CH, NP), jnp.float32),
            pltpu.VMEM((_L,), jnp.float32),
            pltpu.SemaphoreType.DMA,
            pltpu.SemaphoreType.DMA,
            pltpu.SemaphoreType.DMA,
            pltpu.SemaphoreType.DMA,
        ],
        name="otloss_sc",
    )


def kernel(output_probs, target_class, C):
    B, N = output_probs.shape
    NP = -(-N // 128) * 128
    idx3 = target_class.astype(jnp.int32).reshape(_NW, (B // _NW) // _CH, _CH)
    c_pad = jnp.pad(C, ((0, 0), (0, NP - N)))
    partials = _build_sc_call(B, N)(output_probs, idx3, c_pad)
    return jnp.sum(partials) / B


# speedup vs baseline: 1.3771x; 1.0058x over previous
import functools

import jax
import jax.numpy as jnp
from jax import lax
from jax.experimental import pallas as pl
from jax.experimental.pallas import tpu as pltpu
from jax.experimental.pallas import tpu_sc as plsc

_NC = 2
_NS = 16
_NW = _NC * _NS
_L = 16
_CH = 16


@functools.lru_cache(maxsize=None)
def _build_sc_call(B, N):
    NP = -(-N // 128) * 128
    NCH = (B // _NW) // _CH
    FS = N // _L
    REM = N % _L
    TOFF = N - _L

    mesh = plsc.VectorSubcoreMesh(core_axis_name="c", subcore_axis_name="s")

    def body(p_hbm, t_hbm, c_hbm, out_hbm,
             idx_v, pbuf0, pbuf1, gbuf0, gbuf1, obuf,
             lsem0, lsem1, gsem0, gsem1):
        cid = lax.axis_index("c")
        sid = lax.axis_index("s")
        wid = cid * _NS + sid
        zv = jnp.zeros((_L,), jnp.float32)
        lanes = lax.iota(jnp.int32, _L)
        tmask = lanes >= (_L - REM)

        pltpu.sync_copy(t_hbm.at[wid], idx_v)

        rowbase = wid * (NCH * _CH)
        pbufs = (pbuf0, pbuf1)
        gbufs = (gbuf0, gbuf1)
        lsems = (lsem0, lsem1)
        gsems = (gsem0, gsem1)

        def start(j):
            b = j % 2
            ldj = pltpu.async_copy(
                p_hbm.at[pl.ds(rowbase + j * _CH, _CH)], pbufs[b], lsems[b])
            gdj = pltpu.async_copy(
                c_hbm.at[idx_v.at[j]], gbufs[b], gsems[b])
            return ldj, gdj

        NA = 4

        def chunk_dot(pb, gb, acc):
            def fbody(jj, accs):
                off = jj * _L
                accs = list(accs)
                for r in range(_CH):
                    accs[r % NA] = (accs[r % NA]
                                    + pb[r, pl.ds(off, _L)]
                                    * gb[r, pl.ds(off, _L)])
                return tuple(accs)
            accs = lax.fori_loop(0, FS, fbody, (acc,) + (zv,) * (NA - 1))
            accs = list(accs)
            if REM:
                for r in range(_CH):
                    t = pb[r, pl.ds(TOFF, _L)] * gb[r, pl.ds(TOFF, _L)]
                    accs[r % NA] = accs[r % NA] + jnp.where(tmask, t, zv)
            out = accs[0]
            for a in accs[1:]:
                out = out + a
            return out

        acc = zv
        pend = [None, None]
        pend[0] = start(0)
        for j in range(NCH):
            b = j % 2
            ldj, gdj = pend[b]
            ldj.wait()
            gdj.wait()
            if j + 1 < NCH:
                pend[1 - b] = start(j + 1)
            acc = chunk_dot(pbufs[b], gbufs[b], acc)

        obuf[...] = acc
        pltpu.sync_copy(obuf, out_hbm.at[wid])

    return pl.kernel(
        body,
        out_type=jax.ShapeDtypeStruct((_NW, _L), jnp.float32),
        mesh=mesh,
        scratch_types=[
            pltpu.VMEM((NCH, _CH), jnp.int32),
            pltpu.VMEM((_CH, N), jnp.float32),
            pltpu.VMEM((_CH, N), jnp.float32),
            pltpu.VMEM((_CH, NP), jnp.float32),
            pltpu.VMEM((_CH, NP), jnp.float32),
            pltpu.VMEM((_L,), jnp.float32),
            pltpu.SemaphoreType.DMA,
            pltpu.SemaphoreType.DMA,
            pltpu.SemaphoreType.DMA,
            pltpu.SemaphoreType.DMA,
        ],
        name="otloss_sc",
    )


def kernel(output_probs, target_class, C):
    B, N = output_probs.shape
    NP = -(-N // 128) * 128
    idx3 = target_class.astype(jnp.int32).reshape(_NW, (B // _NW) // _CH, _CH)
    c_pad = jnp.pad(C, ((0, 0), (0, NP - N)))
    partials = _build_sc_call(B, N)(output_probs, idx3, c_pad)
    return jnp.sum(partials) / B


# speedup vs baseline: 1.4090x; 1.0231x over previous
import functools

import jax
import jax.numpy as jnp
from jax import lax
from jax.experimental import pallas as pl
from jax.experimental.pallas import tpu as pltpu
from jax.experimental.pallas import tpu_sc as plsc

_NC = 2
_NS = 16
_NW = _NC * _NS
_L = 16
_CH = 16


@functools.lru_cache(maxsize=None)
def _build_sc_call(B, N):
    NP = -(-N // 128) * 128
    NCH = (B // _NW) // _CH
    FS = N // _L
    REM = N % _L
    TOFF = N - _L

    mesh = plsc.VectorSubcoreMesh(core_axis_name="c", subcore_axis_name="s")

    def body(p_hbm, t_hbm, c_hbm, out_hbm,
             idx_v, pbuf0, pbuf1, gbuf0, gbuf1, obuf,
             lsem0, lsem1, gsem0, gsem1):
        cid = lax.axis_index("c")
        sid = lax.axis_index("s")
        wid = cid * _NS + sid
        zv = jnp.zeros((_L,), jnp.float32)
        lanes = lax.iota(jnp.int32, _L)
        tmask = lanes >= (_L - REM)

        pltpu.sync_copy(t_hbm.at[wid], idx_v)

        rowbase = wid * (NCH * _CH)
        pbufs = (pbuf0, pbuf1)
        gbufs = (gbuf0, gbuf1)
        lsems = (lsem0, lsem1)
        gsems = (gsem0, gsem1)

        def start(j):
            b = j % 2
            ldj = pltpu.async_copy(
                p_hbm.at[pl.ds(rowbase + j * _CH, _CH)], pbufs[b], lsems[b])
            gdj = pltpu.async_copy(
                c_hbm.at[idx_v.at[j]], gbufs[b], gsems[b])
            return ldj, gdj

        NA = 4

        def chunk_dot(pb, gb, acc):
            def fbody(jj, accs):
                off = jj * _L
                accs = list(accs)
                for r in range(_CH):
                    accs[r % NA] = (accs[r % NA]
                                    + pb[r, pl.ds(off, _L)]
                                    * gb[r, pl.ds(off, _L)])
                return tuple(accs)
            accs = lax.fori_loop(0, FS, fbody, (acc,) + (zv,) * (NA - 1))
            accs = list(accs)
            if REM:
                for r in range(_CH):
                    t = pb[r, pl.ds(TOFF, _L)] * gb[r, pl.ds(TOFF, _L)]
                    accs[r % NA] = accs[r % NA] + jnp.where(tmask, t, zv)
            out = accs[0]
            for a in accs[1:]:
                out = out + a
            return out

        acc = zv
        pend = [None, None]
        pend[0] = start(0)
        for j in range(NCH):
            b = j % 2
            ldj, gdj = pend[b]
            ldj.wait()
            gdj.wait()
            if j + 1 < NCH:
                pend[1 - b] = start(j + 1)
            acc = chunk_dot(pbufs[b], gbufs[b], acc)

        obuf[...] = acc
        pltpu.sync_copy(obuf, out_hbm.at[wid])

    return pl.kernel(
        body,
        out_type=jax.ShapeDtypeStruct((_NW, _L), jnp.float32),
        mesh=mesh,
        scratch_types=[
            pltpu.VMEM((NCH, _CH), jnp.int32),
            pltpu.VMEM((_CH, N), jnp.float32),
            pltpu.VMEM((_CH, N), jnp.float32),
            pltpu.VMEM((_CH, NP), jnp.float32),
            pltpu.VMEM((_CH, NP), jnp.float32),
            pltpu.VMEM((_L,), jnp.float32),
            pltpu.SemaphoreType.DMA,
            pltpu.SemaphoreType.DMA,
            pltpu.SemaphoreType.DMA,
            pltpu.SemaphoreType.DMA,
        ],
        name="otloss_sc",
    )


_R = 512
_BSC_FRAC = 4


@functools.lru_cache(maxsize=None)
def _build_tc_call(B, N, BSC):
    NB = (B - BSC) // _R
    OFF = BSC // _R

    def tc_body(t_ref, p_ref, c_ref, o_ref):
        t_col = t_ref[0]
        kio = lax.broadcasted_iota(jnp.int32, (_R, N), 1)
        g = (kio == t_col).astype(jnp.bfloat16)
        rows = jnp.dot(g, c_ref[...], preferred_element_type=jnp.float32)
        s = jnp.sum(rows * p_ref[...])

        @pl.when(pl.program_id(0) == 0)
        def _():
            o_ref[0, 0] = 0.0

        o_ref[0, 0] += s

    return pl.pallas_call(
        tc_body,
        grid=(NB,),
        in_specs=[
            pl.BlockSpec((1, _R, 1), lambda i: (i, 0, 0)),
            pl.BlockSpec((_R, N), lambda i: (OFF + i, 0)),
            pl.BlockSpec((N, N), lambda i: (0, 0)),
        ],
        out_specs=pl.BlockSpec(memory_space=pltpu.SMEM),
        out_shape=jax.ShapeDtypeStruct((1, 1), jnp.float32),
        compiler_params=pltpu.CompilerParams(
            dimension_semantics=("arbitrary",)),
        name="otloss_tc",
    )


def kernel(output_probs, target_class, C):
    B, N = output_probs.shape
    NP = -(-N // 128) * 128
    BSC = B // _BSC_FRAC
    tci = target_class.astype(jnp.int32)
    idx3 = tci[:BSC].reshape(_NW, (BSC // _NW) // _CH, _CH)
    c_pad = jnp.pad(C, ((0, 0), (0, NP - N)))
    partials = _build_sc_call(BSC, N)(output_probs, idx3, c_pad)
    t3 = tci[BSC:].reshape((B - BSC) // _R, _R, 1)
    tc_sum = _build_tc_call(B, N, BSC)(t3, output_probs, C.astype(jnp.bfloat16))
    return (jnp.sum(partials) + tc_sum[0, 0]) / B


# speedup vs baseline: 2.1215x; 1.5057x over previous
import functools

import jax
import jax.numpy as jnp
from jax import lax
from jax.experimental import pallas as pl
from jax.experimental.pallas import tpu as pltpu
from jax.experimental.pallas import tpu_sc as plsc

_NC = 2
_NS = 16
_NW = _NC * _NS
_L = 16
_CH = 16


@functools.lru_cache(maxsize=None)
def _build_sc_call(B, N):
    NP = -(-N // 128) * 128
    NCH = (B // _NW) // _CH
    FS = N // _L
    REM = N % _L
    TOFF = N - _L

    mesh = plsc.VectorSubcoreMesh(core_axis_name="c", subcore_axis_name="s")

    def body(p_hbm, t_hbm, c_hbm, out_hbm,
             idx_v, pbuf0, pbuf1, gbuf0, gbuf1, obuf,
             lsem0, lsem1, gsem0, gsem1):
        cid = lax.axis_index("c")
        sid = lax.axis_index("s")
        wid = cid * _NS + sid
        zv = jnp.zeros((_L,), jnp.float32)
        lanes = lax.iota(jnp.int32, _L)
        tmask = lanes >= (_L - REM)

        pltpu.sync_copy(t_hbm.at[wid], idx_v)

        rowbase = wid * (NCH * _CH)
        pbufs = (pbuf0, pbuf1)
        gbufs = (gbuf0, gbuf1)
        lsems = (lsem0, lsem1)
        gsems = (gsem0, gsem1)

        def start(j):
            b = j % 2
            ldj = pltpu.async_copy(
                p_hbm.at[pl.ds(rowbase + j * _CH, _CH)], pbufs[b], lsems[b])
            gdj = pltpu.async_copy(
                c_hbm.at[idx_v.at[j]], gbufs[b], gsems[b])
            return ldj, gdj

        NA = 4

        def chunk_dot(pb, gb, acc):
            def fbody(jj, accs):
                off = jj * _L
                accs = list(accs)
                for r in range(_CH):
                    accs[r % NA] = (accs[r % NA]
                                    + pb[r, pl.ds(off, _L)]
                                    * gb[r, pl.ds(off, _L)])
                return tuple(accs)
            accs = lax.fori_loop(0, FS, fbody, (acc,) + (zv,) * (NA - 1))
            accs = list(accs)
            if REM:
                for r in range(_CH):
                    t = pb[r, pl.ds(TOFF, _L)] * gb[r, pl.ds(TOFF, _L)]
                    accs[r % NA] = accs[r % NA] + jnp.where(tmask, t, zv)
            out = accs[0]
            for a in accs[1:]:
                out = out + a
            return out

        acc = zv
        pend = [None, None]
        pend[0] = start(0)
        for j in range(NCH):
            b = j % 2
            ldj, gdj = pend[b]
            ldj.wait()
            gdj.wait()
            if j + 1 < NCH:
                pend[1 - b] = start(j + 1)
            acc = chunk_dot(pbufs[b], gbufs[b], acc)

        obuf[...] = acc
        pltpu.sync_copy(obuf, out_hbm.at[wid])

    return pl.kernel(
        body,
        out_type=jax.ShapeDtypeStruct((_NW, _L), jnp.float32),
        mesh=mesh,
        scratch_types=[
            pltpu.VMEM((NCH, _CH), jnp.int32),
            pltpu.VMEM((_CH, N), jnp.float32),
            pltpu.VMEM((_CH, N), jnp.float32),
            pltpu.VMEM((_CH, NP), jnp.float32),
            pltpu.VMEM((_CH, NP), jnp.float32),
            pltpu.VMEM((_L,), jnp.float32),
            pltpu.SemaphoreType.DMA,
            pltpu.SemaphoreType.DMA,
            pltpu.SemaphoreType.DMA,
            pltpu.SemaphoreType.DMA,
        ],
        name="otloss_sc",
    )


_R = 512
_BSC_FRAC = 4


@functools.lru_cache(maxsize=None)
def _build_tc_call(B, N, BSC):
    NB = (B - BSC) // _R
    OFF = BSC // _R

    def tc_body(t_ref, pt_ref, ct_ref, o_ref):
        t_row = t_ref[0]
        kio = lax.broadcasted_iota(jnp.int32, (N, _R), 0)
        gt = (kio == t_row).astype(jnp.bfloat16)
        dt = jnp.dot(ct_ref[...], gt, preferred_element_type=jnp.float32)
        s = jnp.sum(dt * pt_ref[...])

        @pl.when(pl.program_id(0) == 0)
        def _():
            o_ref[0, 0] = 0.0

        o_ref[0, 0] += s

    return pl.pallas_call(
        tc_body,
        grid=(NB,),
        in_specs=[
            pl.BlockSpec((1, 1, _R), lambda i: (i, 0, 0)),
            pl.BlockSpec((N, _R), lambda i: (0, OFF + i)),
            pl.BlockSpec((N, N), lambda i: (0, 0)),
        ],
        out_specs=pl.BlockSpec(memory_space=pltpu.SMEM),
        out_shape=jax.ShapeDtypeStruct((1, 1), jnp.float32),
        compiler_params=pltpu.CompilerParams(
            dimension_semantics=("arbitrary",)),
        name="otloss_tc",
    )


def kernel(output_probs, target_class, C):
    B, N = output_probs.shape
    NP = -(-N // 128) * 128
    BSC = B // _BSC_FRAC
    tci = target_class.astype(jnp.int32)
    idx3 = tci[:BSC].reshape(_NW, (BSC // _NW) // _CH, _CH)
    c_pad = jnp.pad(C, ((0, 0), (0, NP - N)))
    partials = _build_sc_call(BSC, N)(output_probs[:BSC], idx3, c_pad)
    t3 = tci[BSC:].reshape((B - BSC) // _R, 1, _R)
    ct_b = C.T.astype(jnp.bfloat16)
    tc_sum = _build_tc_call(B, N, BSC)(t3, output_probs.T, ct_b)
    return (jnp.sum(partials) + tc_sum[0, 0]) / B


# speedup vs baseline: 2.4389x; 1.1496x over previous
import functools

import jax
import jax.numpy as jnp
from jax import lax
from jax.experimental import pallas as pl
from jax.experimental.pallas import tpu as pltpu
from jax.experimental.pallas import tpu_sc as plsc

_NC = 2
_NS = 16
_NW = _NC * _NS
_L = 16
_CH = 16


@functools.lru_cache(maxsize=None)
def _build_sc_call(B, N):
    NP = -(-N // 128) * 128
    NCH = (B // _NW) // _CH
    FS = N // _L
    REM = N % _L
    TOFF = N - _L

    mesh = plsc.VectorSubcoreMesh(core_axis_name="c", subcore_axis_name="s")

    def body(p_hbm, t_hbm, c_hbm, out_hbm,
             idx_v, pbuf0, pbuf1, gbuf0, gbuf1, obuf,
             lsem0, lsem1, gsem0, gsem1):
        cid = lax.axis_index("c")
        sid = lax.axis_index("s")
        wid = cid * _NS + sid
        zv = jnp.zeros((_L,), jnp.float32)
        lanes = lax.iota(jnp.int32, _L)
        tmask = lanes >= (_L - REM)

        pltpu.sync_copy(t_hbm.at[wid], idx_v)

        rowbase = wid * (NCH * _CH)
        pbufs = (pbuf0, pbuf1)
        gbufs = (gbuf0, gbuf1)
        lsems = (lsem0, lsem1)
        gsems = (gsem0, gsem1)

        def start(j):
            b = j % 2
            ldj = pltpu.async_copy(
                p_hbm.at[pl.ds(rowbase + j * _CH, _CH)], pbufs[b], lsems[b])
            gdj = pltpu.async_copy(
                c_hbm.at[idx_v.at[j]], gbufs[b], gsems[b])
            return ldj, gdj

        NA = 4

        def chunk_dot(pb, gb, acc):
            def fbody(jj, accs):
                off = jj * _L
                accs = list(accs)
                for r in range(_CH):
                    accs[r % NA] = (accs[r % NA]
                                    + pb[r, pl.ds(off, _L)]
                                    * gb[r, pl.ds(off, _L)])
                return tuple(accs)
            accs = lax.fori_loop(0, FS, fbody, (acc,) + (zv,) * (NA - 1))
            accs = list(accs)
            if REM:
                for r in range(_CH):
                    t = pb[r, pl.ds(TOFF, _L)] * gb[r, pl.ds(TOFF, _L)]
                    accs[r % NA] = accs[r % NA] + jnp.where(tmask, t, zv)
            out = accs[0]
            for a in accs[1:]:
                out = out + a
            return out

        acc = zv
        pend = [None, None]
        pend[0] = start(0)
        for j in range(NCH):
            b = j % 2
            ldj, gdj = pend[b]
            ldj.wait()
            gdj.wait()
            if j + 1 < NCH:
                pend[1 - b] = start(j + 1)
            acc = chunk_dot(pbufs[b], gbufs[b], acc)

        obuf[...] = acc
        pltpu.sync_copy(obuf, out_hbm.at[wid])

    return pl.kernel(
        body,
        out_type=jax.ShapeDtypeStruct((_NW, _L), jnp.float32),
        mesh=mesh,
        scratch_types=[
            pltpu.VMEM((NCH, _CH), jnp.int32),
            pltpu.VMEM((_CH, N), jnp.float32),
            pltpu.VMEM((_CH, N), jnp.float32),
            pltpu.VMEM((_CH, NP), jnp.float32),
            pltpu.VMEM((_CH, NP), jnp.float32),
            pltpu.VMEM((_L,), jnp.float32),
            pltpu.SemaphoreType.DMA,
            pltpu.SemaphoreType.DMA,
            pltpu.SemaphoreType.DMA,
            pltpu.SemaphoreType.DMA,
        ],
        name="otloss_sc",
    )


_R = 512


@functools.lru_cache(maxsize=None)
def _build_tr_call(B, N, BSC):
    def tr_body(pt_ref, o_ref):
        o_ref[...] = pt_ref[...].T

    return pl.pallas_call(
        tr_body,
        grid=(BSC // _R,),
        in_specs=[pl.BlockSpec((N, _R), lambda i: (0, i))],
        out_specs=pl.BlockSpec((_R, N), lambda i: (i, 0)),
        out_shape=jax.ShapeDtypeStruct((BSC, N), jnp.float32),
        name="otloss_tr",
    )


@functools.lru_cache(maxsize=None)
def _build_tc_call(B, N, BSC):
    NB = (B - BSC) // _R
    OFF = BSC // _R

    def tc_body(t_ref, pt_ref, ct_ref, o_ref):
        t_row = t_ref[0]
        kio = lax.broadcasted_iota(jnp.int32, (N, _R), 0)
        gt = (kio == t_row).astype(jnp.bfloat16)
        dt = jnp.dot(ct_ref[...], gt, preferred_element_type=jnp.float32)
        s = jnp.sum(dt * pt_ref[...])

        @pl.when(pl.program_id(0) == 0)
        def _():
            o_ref[0, 0] = 0.0

        o_ref[0, 0] += s

    return pl.pallas_call(
        tc_body,
        grid=(NB,),
        in_specs=[
            pl.BlockSpec((1, 1, _R), lambda i: (i, 0, 0)),
            pl.BlockSpec((N, _R), lambda i: (0, OFF + i)),
            pl.BlockSpec((N, N), lambda i: (0, 0)),
        ],
        out_specs=pl.BlockSpec(memory_space=pltpu.SMEM),
        out_shape=jax.ShapeDtypeStruct((1, 1), jnp.float32),
        compiler_params=pltpu.CompilerParams(
            dimension_semantics=("arbitrary",)),
        name="otloss_tc",
    )


def kernel(output_probs, target_class, C):
    B, N = output_probs.shape
    NP = -(-N // 128) * 128
    BSC = (3 * B // 8 // 512) * 512
    tci = target_class.astype(jnp.int32)
    idx3 = tci[:BSC].reshape(_NW, (BSC // _NW) // _CH, _CH)
    c_pad = jnp.pad(C, ((0, 0), (0, NP - N)))
    p_sc = _build_tr_call(B, N, BSC)(output_probs.T)
    partials = _build_sc_call(BSC, N)(p_sc, idx3, c_pad)
    t3 = tci[BSC:].reshape((B - BSC) // _R, 1, _R)
    ct_b = C.T.astype(jnp.bfloat16)
    tc_sum = _build_tc_call(B, N, BSC)(t3, output_probs.T, ct_b)
    return (jnp.sum(partials) + tc_sum[0, 0]) / B


# speedup vs baseline: 2.4479x; 1.0037x over previous
import functools

import jax
import jax.numpy as jnp
from jax import lax
from jax.experimental import pallas as pl
from jax.experimental.pallas import tpu as pltpu
from jax.experimental.pallas import tpu_sc as plsc

_NC = 2
_NS = 16
_NW = _NC * _NS
_L = 16
_CH = 16


@functools.lru_cache(maxsize=None)
def _build_sc_call(B, N):
    NP = -(-N // 128) * 128
    NCH = (B // _NW) // _CH
    FS = N // _L
    REM = N % _L
    TOFF = N - _L

    mesh = plsc.VectorSubcoreMesh(core_axis_name="c", subcore_axis_name="s")

    def body(p_hbm, t_hbm, c_hbm, out_hbm,
             idx_v, pbuf0, pbuf1, gbuf0, gbuf1, obuf,
             lsem0, lsem1, gsem0, gsem1):
        cid = lax.axis_index("c")
        sid = lax.axis_index("s")
        wid = cid * _NS + sid
        zv = jnp.zeros((_L,), jnp.float32)
        lanes = lax.iota(jnp.int32, _L)
        tmask = lanes >= (_L - REM)

        pltpu.sync_copy(t_hbm.at[wid], idx_v)

        rowbase = wid * (NCH * _CH)
        pbufs = (pbuf0, pbuf1)
        gbufs = (gbuf0, gbuf1)
        lsems = (lsem0, lsem1)
        gsems = (gsem0, gsem1)

        def start(j):
            b = j % 2
            ldj = pltpu.async_copy(
                p_hbm.at[pl.ds(rowbase + j * _CH, _CH)], pbufs[b], lsems[b])
            gdj = pltpu.async_copy(
                c_hbm.at[idx_v.at[j]], gbufs[b], gsems[b])
            return ldj, gdj

        NA = 4

        def chunk_dot(pb, gb, acc):
            def fbody(jj, accs):
                off = jj * _L
                accs = list(accs)
                for r in range(_CH):
                    accs[r % NA] = (accs[r % NA]
                                    + pb[r, pl.ds(off, _L)]
                                    * gb[r, pl.ds(off, _L)])
                return tuple(accs)
            accs = lax.fori_loop(0, FS, fbody, (acc,) + (zv,) * (NA - 1))
            accs = list(accs)
            if REM:
                for r in range(_CH):
                    t = pb[r, pl.ds(TOFF, _L)] * gb[r, pl.ds(TOFF, _L)]
                    accs[r % NA] = accs[r % NA] + jnp.where(tmask, t, zv)
            out = accs[0]
            for a in accs[1:]:
                out = out + a
            return out

        acc = zv
        pend = [None, None]
        pend[0] = start(0)
        for j in range(NCH):
            b = j % 2
            ldj, gdj = pend[b]
            ldj.wait()
            gdj.wait()
            if j + 1 < NCH:
                pend[1 - b] = start(j + 1)
            acc = chunk_dot(pbufs[b], gbufs[b], acc)

        obuf[...] = acc
        pltpu.sync_copy(obuf, out_hbm.at[wid])

    return pl.kernel(
        body,
        out_type=jax.ShapeDtypeStruct((_NW, _L), jnp.float32),
        mesh=mesh,
        scratch_types=[
            pltpu.VMEM((NCH, _CH), jnp.int32),
            pltpu.VMEM((_CH, N), jnp.float32),
            pltpu.VMEM((_CH, N), jnp.float32),
            pltpu.VMEM((_CH, NP), jnp.float32),
            pltpu.VMEM((_CH, NP), jnp.float32),
            pltpu.VMEM((_L,), jnp.float32),
            pltpu.SemaphoreType.DMA,
            pltpu.SemaphoreType.DMA,
            pltpu.SemaphoreType.DMA,
            pltpu.SemaphoreType.DMA,
        ],
        name="otloss_sc",
    )


_R = 512


@functools.lru_cache(maxsize=None)
def _build_tr_call(B, N, BSC):
    def tr_body(pt_ref, o_ref):
        o_ref[...] = pt_ref[...].T

    return pl.pallas_call(
        tr_body,
        grid=(BSC // _R,),
        in_specs=[pl.BlockSpec((N, _R), lambda i: (0, i))],
        out_specs=pl.BlockSpec((_R, N), lambda i: (i, 0)),
        out_shape=jax.ShapeDtypeStruct((BSC, N), jnp.float32),
        name="otloss_tr",
    )


@functools.lru_cache(maxsize=None)
def _build_tc_call(B, N, BSC):
    NB = (B - BSC) // _R
    OFF = BSC // _R

    def tc_body(t_ref, pt_ref, ct_ref, o_ref):
        t_row = t_ref[0]
        kio = lax.broadcasted_iota(jnp.int32, (N, _R), 0)
        gt = (kio == t_row).astype(jnp.bfloat16)
        dt = jnp.dot(ct_ref[...], gt, preferred_element_type=jnp.float32)
        s = jnp.sum(dt * pt_ref[...])

        @pl.when(pl.program_id(0) == 0)
        def _():
            o_ref[0, 0] = 0.0

        o_ref[0, 0] += s

    return pl.pallas_call(
        tc_body,
        grid=(NB,),
        in_specs=[
            pl.BlockSpec((1, 1, _R), lambda i: (i, 0, 0)),
            pl.BlockSpec((N, _R), lambda i: (0, OFF + i)),
            pl.BlockSpec((N, N), lambda i: (0, 0)),
        ],
        out_specs=pl.BlockSpec(memory_space=pltpu.SMEM),
        out_shape=jax.ShapeDtypeStruct((1, 1), jnp.float32),
        compiler_params=pltpu.CompilerParams(
            dimension_semantics=("arbitrary",)),
        name="otloss_tc",
    )


def kernel(output_probs, target_class, C):
    B, N = output_probs.shape
    NP = -(-N // 128) * 128
    BSC = 13 * (B // 32)
    tci = target_class.astype(jnp.int32)
    idx3 = tci[:BSC].reshape(_NW, (BSC // _NW) // _CH, _CH)
    c_pad = jnp.pad(C, ((0, 0), (0, NP - N)))
    p_sc = _build_tr_call(B, N, BSC)(output_probs.T)
    partials = _build_sc_call(BSC, N)(p_sc, idx3, c_pad)
    t3 = tci[BSC:].reshape((B - BSC) // _R, 1, _R)
    ct_b = C.T.astype(jnp.bfloat16)
    tc_sum = _build_tc_call(B, N, BSC)(t3, output_probs.T, ct_b)
    return (jnp.sum(partials) + tc_sum[0, 0]) / B


# speedup vs baseline: 2.5621x; 1.0467x over previous
import functools

import jax
import jax.numpy as jnp
from jax import lax
from jax.experimental import pallas as pl
from jax.experimental.pallas import tpu as pltpu
from jax.experimental.pallas import tpu_sc as plsc

_NC = 2
_NS = 16
_NW = _NC * _NS
_L = 16
_CH = 16


@functools.lru_cache(maxsize=None)
def _build_sc_call(B, N):
    NP = -(-N // 128) * 128
    NCH = (B // _NW) // _CH
    FS = N // _L
    REM = N % _L
    TOFF = N - _L

    mesh = plsc.VectorSubcoreMesh(core_axis_name="c", subcore_axis_name="s")

    def body(p_hbm, t_hbm, c_hbm, out_hbm,
             idx_v, pbuf0, pbuf1, gbuf0, gbuf1, obuf,
             lsem0, lsem1, gsem0, gsem1):
        cid = lax.axis_index("c")
        sid = lax.axis_index("s")
        wid = cid * _NS + sid
        zv = jnp.zeros((_L,), jnp.float32)
        lanes = lax.iota(jnp.int32, _L)
        tmask = lanes >= (_L - REM)

        pltpu.sync_copy(t_hbm.at[wid], idx_v)

        rowbase = wid * (NCH * _CH)
        pbufs = (pbuf0, pbuf1)
        gbufs = (gbuf0, gbuf1)
        lsems = (lsem0, lsem1)
        gsems = (gsem0, gsem1)

        def start(j):
            b = j % 2
            ldj = pltpu.async_copy(
                p_hbm.at[pl.ds(rowbase + j * _CH, _CH)], pbufs[b], lsems[b])
            gdj = pltpu.async_copy(
                c_hbm.at[idx_v.at[j]], gbufs[b], gsems[b])
            return ldj, gdj

        NA = 4

        def chunk_dot(pb, gb, acc):
            def fbody(jj, accs):
                off = jj * _L
                accs = list(accs)
                for r in range(_CH):
                    accs[r % NA] = (accs[r % NA]
                                    + pb[r, pl.ds(off, _L)]
                                    * gb[r, pl.ds(off, _L)])
                return tuple(accs)
            accs = lax.fori_loop(0, FS, fbody, (acc,) + (zv,) * (NA - 1))
            accs = list(accs)
            if REM:
                for r in range(_CH):
                    t = pb[r, pl.ds(TOFF, _L)] * gb[r, pl.ds(TOFF, _L)]
                    accs[r % NA] = accs[r % NA] + jnp.where(tmask, t, zv)
            out = accs[0]
            for a in accs[1:]:
                out = out + a
            return out

        acc = zv
        pend = [None, None]
        pend[0] = start(0)
        for j in range(NCH):
            b = j % 2
            ldj, gdj = pend[b]
            ldj.wait()
            gdj.wait()
            if j + 1 < NCH:
                pend[1 - b] = start(j + 1)
            acc = chunk_dot(pbufs[b], gbufs[b], acc)

        obuf[...] = acc
        pltpu.sync_copy(obuf, out_hbm.at[wid])

    return pl.kernel(
        body,
        out_type=jax.ShapeDtypeStruct((_NW, _L), jnp.float32),
        mesh=mesh,
        scratch_types=[
            pltpu.VMEM((NCH, _CH), jnp.int32),
            pltpu.VMEM((_CH, N), jnp.float32),
            pltpu.VMEM((_CH, N), jnp.float32),
            pltpu.VMEM((_CH, NP), jnp.float32),
            pltpu.VMEM((_CH, NP), jnp.float32),
            pltpu.VMEM((_L,), jnp.float32),
            pltpu.SemaphoreType.DMA,
            pltpu.SemaphoreType.DMA,
            pltpu.SemaphoreType.DMA,
            pltpu.SemaphoreType.DMA,
        ],
        name="otloss_sc",
    )


_R = 1024


@functools.lru_cache(maxsize=None)
def _build_tr_call(B, N, BSC):
    def tr_body(pt_ref, o_ref):
        o_ref[...] = pt_ref[...].T

    return pl.pallas_call(
        tr_body,
        grid=(BSC // _R,),
        in_specs=[pl.BlockSpec((N, _R), lambda i: (0, i))],
        out_specs=pl.BlockSpec((_R, N), lambda i: (i, 0)),
        out_shape=jax.ShapeDtypeStruct((BSC, N), jnp.float32),
        name="otloss_tr",
    )


@functools.lru_cache(maxsize=None)
def _build_tc_call(B, N, BSC):
    NB = (B - BSC) // _R
    OFF = BSC // _R

    def tc_body(t_ref, pt_ref, ct_ref, o_ref):
        t_row = t_ref[0]
        kio = lax.broadcasted_iota(jnp.int32, (N, _R), 0)
        gt = (kio == t_row).astype(jnp.bfloat16)
        dt = jnp.dot(ct_ref[...], gt, preferred_element_type=jnp.float32)
        s = jnp.sum(dt * pt_ref[...])

        @pl.when(pl.program_id(0) == 0)
        def _():
            o_ref[0, 0] = 0.0

        o_ref[0, 0] += s

    return pl.pallas_call(
        tc_body,
        grid=(NB,),
        in_specs=[
            pl.BlockSpec((1, 1, _R), lambda i: (i, 0, 0)),
            pl.BlockSpec((N, _R), lambda i: (0, OFF + i)),
            pl.BlockSpec((N, N), lambda i: (0, 0)),
        ],
        out_specs=pl.BlockSpec(memory_space=pltpu.SMEM),
        out_shape=jax.ShapeDtypeStruct((1, 1), jnp.float32),
        compiler_params=pltpu.CompilerParams(
            dimension_semantics=("arbitrary",)),
        name="otloss_tc",
    )


def kernel(output_probs, target_class, C):
    B, N = output_probs.shape
    NP = -(-N // 128) * 128
    BSC = 6 * (B // 16)
    tci = target_class.astype(jnp.int32)
    idx3 = tci[:BSC].reshape(_NW, (BSC // _NW) // _CH, _CH)
    c_pad = jnp.pad(C, ((0, 0), (0, NP - N)))
    p_sc = _build_tr_call(B, N, BSC)(output_probs.T)
    partials = _build_sc_call(BSC, N)(p_sc, idx3, c_pad)
    t3 = tci[BSC:].reshape((B - BSC) // _R, 1, _R)
    ct_b = C.T.astype(jnp.bfloat16)
    tc_sum = _build_tc_call(B, N, BSC)(t3, output_probs.T, ct_b)
    return (jnp.sum(partials) + tc_sum[0, 0]) / B
